# Initial kernel scaffold; baseline (speedup 1.0000x reference)
import jax
import jax.numpy as jnp
from jax.experimental import pallas as pl


def kernel(pos, idx):
    raise NotImplementedError("write your pallas kernel here")


# speedup vs baseline: 3.2861x; 3.2861x over previous
import functools

import jax
import jax.numpy as jnp
from jax import lax
from jax.experimental import pallas as pl
from jax.experimental.pallas import tpu as pltpu
from jax.experimental.pallas import tpu_sc as plsc

N = 320000
D = 128
S = 10000

L = 16
NJ = D // L
NW = 32
SPW = 320
S_LAST = S - (NW - 1) * SPW
C = 256

_mesh = plsc.VectorSubcoreMesh(core_axis_name="c", subcore_axis_name="s")


@functools.partial(
    pl.kernel,
    out_type=(
        jax.ShapeDtypeStruct((N, D), jnp.float32),
        jax.ShapeDtypeStruct((S,), jnp.float32),
    ),
    mesh=_mesh,
    compiler_params=pltpu.CompilerParams(needs_layout_passes=False),
    scratch_types=[
        pltpu.VMEM((C, D), jnp.float32),
        pltpu.VMEM((C + L,), jnp.int32),
        pltpu.VMEM((48,), jnp.int32),
        pltpu.VMEM((SPW * D,), jnp.float32),
        pltpu.VMEM((SPW + L,), jnp.float32),
    ],
)
def _seg_unit_norm(pos_hbm, idx_hbm, bounds_hbm, out_hbm, diam_hbm,
                   row_v, idx_v, bounds_v, mean_v, diam_v):
    w = lax.axis_index("s") * 2 + lax.axis_index("c")

    def _sload(ref, i):
        return ref[pl.ds(i, L)][0]

    pltpu.sync_copy(bounds_hbm, bounds_v)
    r0 = _sload(bounds_v, w)
    r1 = _sload(bounds_v, w + 1)
    seg0 = w * SPW
    seg_end = jnp.minimum(seg0 + SPW, S)

    inf = jnp.float32(jnp.inf)
    id_min = jnp.full((L,), inf, jnp.float32)
    id_max = jnp.full((L,), -inf, jnp.float32)
    id_sum = jnp.zeros((L,), jnp.float32)

    def flush_one(cur, cnt, mins, maxs, sums):
        ls = cur - seg0
        ones = jnp.ones((L,), jnp.float32)
        rcp = ones / jnp.broadcast_to(jnp.maximum(cnt, jnp.float32(1.0)), (L,))
        for j in range(NJ):
            mean_v[pl.ds(ls * D + j * L, L)] = sums[j] * rcp
        dv = maxs[0] - mins[0]
        for j in range(1, NJ):
            dv = jnp.maximum(dv, maxs[j] - mins[j])
        diam_v[pl.ds(ls, L)] = jnp.broadcast_to(jnp.max(dv), (L,))

    def close_to(target, carry):
        cur, cnt, mins, maxs, sums = carry

        def flush_branch(_):
            flush_one(cur, cnt, mins, maxs, sums)

            def empty_body(t, z):
                ls = t - seg0
                for j in range(NJ):
                    mean_v[pl.ds(ls * D + j * L, L)] = id_sum
                diam_v[pl.ds(ls, L)] = jnp.full((L,), -inf, jnp.float32)
                return z

            lax.fori_loop(cur + 1, target, empty_body, 0)
            return (target, jnp.float32(0.0),
                    (id_min,) * NJ, (id_max,) * NJ, (id_sum,) * NJ)

        def skip_branch(_):
            return carry

        return lax.cond(cur < target, flush_branch, skip_branch, 0)

    def row_step(i, carry):
        s_val = _sload(idx_v, i)
        cur, cnt, mins, maxs, sums = close_to(s_val, carry)
        rows = [row_v[i, pl.ds(j * L, L)] for j in range(NJ)]
        mins = tuple(jnp.minimum(m, r) for m, r in zip(mins, rows))
        maxs = tuple(jnp.maximum(m, r) for m, r in zip(maxs, rows))
        sums = tuple(s + r for s, r in zip(sums, rows))
        return (cur, cnt + jnp.float32(1.0), mins, maxs, sums)

    k_lo = r0 // C
    k_hi = (r1 + C - 1) // C

    def chunk1_body(k, carry):
        base = k * C
        pltpu.sync_copy(pos_hbm.at[pl.ds(base, C)], row_v)
        pltpu.sync_copy(idx_hbm.at[pl.ds(base, C)], idx_v.at[pl.ds(0, C)])
        lo = jnp.maximum(r0 - base, 0)
        hi = jnp.minimum(r1 - base, C)
        return lax.fori_loop(lo, hi, row_step, carry)

    carry0 = (seg0, jnp.float32(0.0),
              (id_min,) * NJ, (id_max,) * NJ, (id_sum,) * NJ)
    carry = lax.fori_loop(k_lo, k_hi, chunk1_body, carry0)
    close_to(seg_end, carry)

    @pl.when(w < NW - 1)
    def _():
        pltpu.sync_copy(diam_v.at[pl.ds(0, SPW)], diam_hbm.at[pl.ds(seg0, SPW)])

    @pl.when(w == NW - 1)
    def _():
        pltpu.sync_copy(diam_v.at[pl.ds(0, S_LAST)],
                        diam_hbm.at[pl.ds((NW - 1) * SPW, S_LAST)])

    def chunk2_body(k, _):
        base = k * C
        pltpu.sync_copy(pos_hbm.at[pl.ds(base, C)], row_v)
        pltpu.sync_copy(idx_hbm.at[pl.ds(base, C)], idx_v.at[pl.ds(0, C)])
        lo = jnp.maximum(r0 - base, 0)
        hi = jnp.minimum(r1 - base, C)

        def norm_row(i, carry):
            ls = _sload(idx_v, i) - seg0
            svec = jnp.ones((L,), jnp.float32) / (
                diam_v[pl.ds(ls, L)] + jnp.float32(0.01))
            scale = jnp.broadcast_to(svec[0], (L,))
            for j in range(NJ):
                sl = pl.ds(j * L, L)
                m = mean_v[pl.ds(ls * D + j * L, L)]
                row_v[i, sl] = (row_v[i, sl] - m) * scale
            return carry

        lax.fori_loop(lo, hi, norm_row, 0)

        full = jnp.logical_and(lo == 0, hi == C)

        @pl.when(full)
        def _():
            pltpu.sync_copy(row_v, out_hbm.at[pl.ds(base, C)])

        @pl.when(jnp.logical_not(full))
        def _():
            def wr(i, carry):
                pltpu.sync_copy(row_v.at[i], out_hbm.at[base + i])
                return carry

            lax.fori_loop(lo, hi, wr, 0)

        return 0

    lax.fori_loop(k_lo, k_hi, chunk2_body, 0)


def kernel(pos, idx):
    seg_edges = jnp.minimum(
        jnp.arange(NW + 1, dtype=jnp.int32) * SPW, S).astype(jnp.int32)
    bounds = jnp.searchsorted(idx, seg_edges, side="left").astype(jnp.int32)
    bounds = jnp.concatenate([bounds, jnp.zeros((15,), jnp.int32)])
    pos_out, diam = _seg_unit_norm(pos, idx, bounds)
    return (pos_out, diam)


# speedup vs baseline: 9.9075x; 3.0149x over previous
import functools

import jax
import jax.numpy as jnp
from jax import lax
from jax.experimental import pallas as pl
from jax.experimental.pallas import tpu as pltpu
from jax.experimental.pallas import tpu_sc as plsc

N = 320000
D = 128
S = 10000

L = 16
NJ = D // L
NW = 32
SPW = 320
S_LAST = S - (NW - 1) * SPW
C = 256
CI = 2000
NRUN = SPW + 2 * L

_mesh = plsc.VectorSubcoreMesh(core_axis_name="c", subcore_axis_name="s")


@functools.partial(
    pl.kernel,
    out_type=(
        jax.ShapeDtypeStruct((N, D), jnp.float32),
        jax.ShapeDtypeStruct((S,), jnp.float32),
    ),
    mesh=_mesh,
    compiler_params=pltpu.CompilerParams(needs_layout_passes=False),
    scratch_types=[
        pltpu.VMEM((C, D), jnp.float32),
        pltpu.VMEM((CI + L,), jnp.int32),
        pltpu.VMEM((48,), jnp.int32),
        pltpu.VMEM((SPW * D,), jnp.float32),
        pltpu.VMEM((SPW + L,), jnp.float32),
        pltpu.VMEM((NRUN,), jnp.int32),
        pltpu.VMEM((NRUN,), jnp.int32),
    ],
)
def _seg_unit_norm(pos_hbm, idx_hbm, bounds_hbm, out_hbm, diam_hbm,
                   row_v, ibuf, bounds_v, mean_v, diam_v, bnd_v, sid_v):
    w = lax.axis_index("s") * 2 + lax.axis_index("c")

    def _sload(ref, i):
        return ref[pl.ds(i, L)][0]

    pltpu.sync_copy(bounds_hbm, bounds_v)
    r0 = _sload(bounds_v, w)
    r1 = _sload(bounds_v, w + 1)
    seg0 = w * SPW

    inf = jnp.float32(jnp.inf)
    ones = jnp.ones((L,), jnp.float32)
    lane_iota = lax.iota(jnp.int32, L)
    lane0 = lane_iota == 0
    id_min = jnp.full((L,), inf, jnp.float32)
    id_max = jnp.full((L,), -inf, jnp.float32)
    id_sum = jnp.zeros((L,), jnp.float32)
    id_accs = ((id_min,) * NJ, (id_max,) * NJ, (id_sum,) * NJ)

    def init_diam(t, z):
        diam_v[pl.ds(t * L, L)] = id_max
        return z

    lax.fori_loop(0, (SPW + L) // L, init_diam, 0)

    def p0_chunk(k, wpos):
        base = k * CI
        pltpu.sync_copy(idx_hbm.at[pl.ds(base, CI)], ibuf.at[pl.ds(L, CI)])

        @pl.when(k > 0)
        def _():
            pltpu.sync_copy(idx_hbm.at[pl.ds(base - L, L)],
                            ibuf.at[pl.ds(0, L)])

        @pl.when(k == 0)
        def _():
            ibuf[pl.ds(0, L)] = jnp.full((L,), -1, jnp.int32)

        def group(g, wp):
            off = L + g * L
            v = ibuf[pl.ds(off, L)]
            p = ibuf[pl.ds(off - 1, L)]
            rowv = jnp.broadcast_to(base + g * L, (L,)) + lane_iota
            m = (v != p) & (rowv >= r0) & (rowv < r1)
            plsc.store_compressed(bnd_v.at[pl.ds(wp, L)], rowv, mask=m)
            plsc.store_compressed(sid_v.at[pl.ds(wp, L)], v, mask=m)
            return wp + plsc.all_reduce_population_count(m)[0]

        return lax.fori_loop(0, CI // L, group, wpos)

    nruns = lax.fori_loop(r0 // CI, (r1 + CI - 1) // CI, p0_chunk, 0)
    bnd_v[pl.ds(nruns, L)] = jnp.broadcast_to(r1, (L,))

    def find_hi(lim, lo0):
        def bs(t, lohi):
            lo, hi = lohi
            mid = (lo + hi) // 2
            c = _sload(bnd_v, mid) < lim
            return (jnp.where(c, mid + 1, lo), jnp.where(c, hi, mid))

        lo, _ = lax.fori_loop(0, 9, bs, (lo0, nruns))
        return jnp.minimum(lo, nruns)

    k_lo = r0 // C
    k_hi = (r1 + C - 1) // C

    def flush(s, accs, cnt):
        mins, maxs, sums = accs
        ls = s - seg0
        cntv = jnp.broadcast_to(cnt, (L,)).astype(jnp.float32)
        rcp = ones / jnp.maximum(cntv, ones)
        for j in range(NJ):
            mean_v[pl.ds(ls * D + j * L, L)] = sums[j] * rcp
        dv = maxs[0] - mins[0]
        for j in range(1, NJ):
            dv = jnp.maximum(dv, maxs[j] - mins[j])
        dred = jnp.broadcast_to(jnp.max(dv), (L,))
        plsc.store_scatter(diam_v, [jnp.broadcast_to(ls, (L,))], dred,
                           mask=lane0)

    def p1_chunk(k, carry):
        r_cur, accs = carry
        base = k * C
        lim = base + C
        pltpu.sync_copy(pos_hbm.at[pl.ds(base, C)], row_v)
        r_hi = find_hi(lim, r_cur)

        def run_body(r, accs_):
            b0 = _sload(bnd_v, r)
            b1 = _sload(bnd_v, r + 1)
            rs = jnp.maximum(b0 - base, 0)
            re = jnp.minimum(b1 - base, C)

            def rowacc(i, a):
                mins, maxs, sums = a
                rows = [row_v[i, pl.ds(j * L, L)] for j in range(NJ)]
                mins = tuple(jnp.minimum(m, x) for m, x in zip(mins, rows))
                maxs = tuple(jnp.maximum(m, x) for m, x in zip(maxs, rows))
                sums = tuple(s + x for s, x in zip(sums, rows))
                return (mins, maxs, sums)

            accs_ = lax.fori_loop(rs, re, rowacc, accs_)

            def fin(_):
                flush(_sload(sid_v, r), accs_, b1 - b0)
                return id_accs

            def keep(_):
                return accs_

            return lax.cond(b1 <= lim, fin, keep, 0)

        accs = lax.fori_loop(r_cur, r_hi, run_body, accs)
        r_next = jnp.where(_sload(bnd_v, r_hi) > lim, r_hi - 1, r_hi)
        return (r_next, accs)

    lax.fori_loop(k_lo, k_hi, p1_chunk, (0, id_accs))

    @pl.when(w < NW - 1)
    def _():
        pltpu.sync_copy(diam_v.at[pl.ds(0, SPW)],
                        diam_hbm.at[pl.ds(seg0, SPW)])

    @pl.when(w == NW - 1)
    def _():
        pltpu.sync_copy(diam_v.at[pl.ds(0, S_LAST)],
                        diam_hbm.at[pl.ds((NW - 1) * SPW, S_LAST)])

    def p2_chunk(k, r_cur):
        base = k * C
        lim = base + C
        pltpu.sync_copy(pos_hbm.at[pl.ds(base, C)], row_v)
        r_hi = find_hi(lim, r_cur)

        def run_body(r, z):
            b0 = _sload(bnd_v, r)
            b1 = _sload(bnd_v, r + 1)
            ls = _sload(sid_v, r) - seg0
            svec = ones / (diam_v[pl.ds(ls, L)] + jnp.float32(0.01))
            scale = jnp.broadcast_to(svec[0], (L,))
            means = [mean_v[pl.ds(ls * D + j * L, L)] for j in range(NJ)]
            rs = jnp.maximum(b0 - base, 0)
            re = jnp.minimum(b1 - base, C)

            def rownorm(i, zz):
                for j in range(NJ):
                    sl = pl.ds(j * L, L)
                    row_v[i, sl] = (row_v[i, sl] - means[j]) * scale
                return zz

            lax.fori_loop(rs, re, rownorm, 0)
            return z

        lax.fori_loop(r_cur, r_hi, run_body, 0)

        full = jnp.logical_and(r0 <= base, lim <= r1)

        @pl.when(full)
        def _():
            pltpu.sync_copy(row_v, out_hbm.at[pl.ds(base, C)])

        @pl.when(jnp.logical_not(full))
        def _():
            def wr(i, zz):
                pltpu.sync_copy(row_v.at[i], out_hbm.at[base + i])
                return zz

            lax.fori_loop(jnp.maximum(r0 - base, 0),
                          jnp.minimum(r1 - base, C), wr, 0)

        return jnp.where(_sload(bnd_v, r_hi) > lim, r_hi - 1, r_hi)

    lax.fori_loop(k_lo, k_hi, p2_chunk, 0)


def kernel(pos, idx):
    seg_edges = jnp.minimum(
        jnp.arange(NW + 1, dtype=jnp.int32) * SPW, S).astype(jnp.int32)
    bounds = jnp.searchsorted(idx, seg_edges, side="left").astype(jnp.int32)
    bounds = jnp.concatenate([bounds, jnp.zeros((15,), jnp.int32)])
    pos_out, diam = _seg_unit_norm(pos, idx, bounds)
    return (pos_out, diam)


# speedup vs baseline: 11.7824x; 1.1892x over previous
import functools

import jax
import jax.numpy as jnp
from jax import lax
from jax.experimental import pallas as pl
from jax.experimental.pallas import tpu as pltpu
from jax.experimental.pallas import tpu_sc as plsc

N = 320000
D = 128
S = 10000

L = 16
NJ = D // L
NW = 32
SPW = 320
S_LAST = S - (NW - 1) * SPW
C = 256
CI = 2000
NRUN = SPW + 2 * L

_mesh = plsc.VectorSubcoreMesh(core_axis_name="c", subcore_axis_name="s")


@functools.partial(
    pl.kernel,
    out_type=(
        jax.ShapeDtypeStruct((N, D), jnp.float32),
        jax.ShapeDtypeStruct((S,), jnp.float32),
    ),
    mesh=_mesh,
    compiler_params=pltpu.CompilerParams(needs_layout_passes=False),
    scratch_types=[
        pltpu.VMEM((C, D), jnp.float32),
        pltpu.VMEM((CI + L,), jnp.int32),
        pltpu.VMEM((48,), jnp.int32),
        pltpu.VMEM((SPW * D,), jnp.float32),
        pltpu.VMEM((SPW + L,), jnp.float32),
        pltpu.VMEM((NRUN,), jnp.int32),
        pltpu.VMEM((NRUN,), jnp.int32),
    ],
)
def _seg_unit_norm(pos_hbm, idx_hbm, bounds_hbm, out_hbm, diam_hbm,
                   row_v, ibuf, bounds_v, mean_v, diam_v, bnd_v, sid_v):
    w = lax.axis_index("s") * 2 + lax.axis_index("c")

    def _sload(ref, i):
        return ref[pl.ds(i, L)][0]

    pltpu.sync_copy(bounds_hbm, bounds_v)
    r0 = _sload(bounds_v, w)
    r1 = _sload(bounds_v, w + 1)
    seg0 = w * SPW

    inf = jnp.float32(jnp.inf)
    ones = jnp.ones((L,), jnp.float32)
    lane_iota = lax.iota(jnp.int32, L)
    lane0 = lane_iota == 0
    id_min = jnp.full((L,), inf, jnp.float32)
    id_max = jnp.full((L,), -inf, jnp.float32)
    id_sum = jnp.zeros((L,), jnp.float32)
    id_accs = ((id_min,) * NJ, (id_max,) * NJ, (id_sum,) * NJ)

    def init_diam(t, z):
        diam_v[pl.ds(t * L, L)] = id_max
        return z

    lax.fori_loop(0, (SPW + L) // L, init_diam, 0)

    def p0_chunk(k, wpos):
        base = k * CI
        pltpu.sync_copy(idx_hbm.at[pl.ds(base, CI)], ibuf.at[pl.ds(L, CI)])

        @pl.when(k > 0)
        def _():
            pltpu.sync_copy(idx_hbm.at[pl.ds(base - L, L)],
                            ibuf.at[pl.ds(0, L)])

        @pl.when(k == 0)
        def _():
            ibuf[pl.ds(0, L)] = jnp.full((L,), -1, jnp.int32)

        def group(g, wp):
            off = L + g * L
            v = ibuf[pl.ds(off, L)]
            p = ibuf[pl.ds(off - 1, L)]
            rowv = jnp.broadcast_to(base + g * L, (L,)) + lane_iota
            m = (v != p) & (rowv >= r0) & (rowv < r1)
            plsc.store_compressed(bnd_v.at[pl.ds(wp, L)], rowv, mask=m)
            plsc.store_compressed(sid_v.at[pl.ds(wp, L)], v, mask=m)
            return wp + plsc.all_reduce_population_count(m)[0]

        return lax.fori_loop(0, CI // L, group, wpos)

    nruns = lax.fori_loop(r0 // CI, (r1 + CI - 1) // CI, p0_chunk, 0)
    bnd_v[pl.ds(nruns, L)] = jnp.broadcast_to(r1, (L,))

    def find_hi(lim, lo0):
        def bs(t, lohi):
            lo, hi = lohi
            mid = (lo + hi) // 2
            c = _sload(bnd_v, mid) < lim
            return (jnp.where(c, mid + 1, lo), jnp.where(c, hi, mid))

        lo, _ = lax.fori_loop(0, 9, bs, (lo0, nruns))
        return jnp.minimum(lo, nruns)

    k_lo = r0 // C
    k_hi = (r1 + C - 1) // C

    def flush(s, accs, cnt):
        mins, maxs, sums = accs
        ls = s - seg0
        cntv = jnp.broadcast_to(cnt, (L,)).astype(jnp.float32)
        rcp = ones / jnp.maximum(cntv, ones)
        for j in range(NJ):
            mean_v[pl.ds(ls * D + j * L, L)] = sums[j] * rcp
        dv = maxs[0] - mins[0]
        for j in range(1, NJ):
            dv = jnp.maximum(dv, maxs[j] - mins[j])
        dred = jnp.broadcast_to(jnp.max(dv), (L,))
        plsc.store_scatter(diam_v, [jnp.broadcast_to(ls, (L,))], dred,
                           mask=lane0)

    def p1_chunk(k, carry):
        r_cur, accs = carry
        base = k * C
        lim = base + C
        pltpu.sync_copy(pos_hbm.at[pl.ds(base, C)], row_v)
        r_hi = find_hi(lim, r_cur)

        def run_body(r, accs_):
            b0 = _sload(bnd_v, r)
            b1 = _sload(bnd_v, r + 1)
            rs = jnp.maximum(b0 - base, 0)
            re = jnp.minimum(b1 - base, C)

            def rowacc(i, a):
                mins, maxs, sums = a
                rows = [row_v[i, pl.ds(j * L, L)] for j in range(NJ)]
                mins = tuple(jnp.minimum(m, x) for m, x in zip(mins, rows))
                maxs = tuple(jnp.maximum(m, x) for m, x in zip(maxs, rows))
                sums = tuple(s + x for s, x in zip(sums, rows))
                return (mins, maxs, sums)

            def rowacc_pl(i, a):
                return rowacc(i, a)

            accs_ = plsc.parallel_loop(rs, re, unroll=2,
                                       carry=accs_)(rowacc_pl)

            def fin(_):
                flush(_sload(sid_v, r), accs_, b1 - b0)
                return id_accs

            def keep(_):
                return accs_

            return lax.cond(b1 <= lim, fin, keep, 0)

        accs = lax.fori_loop(r_cur, r_hi, run_body, accs)
        r_next = jnp.where(_sload(bnd_v, r_hi) > lim, r_hi - 1, r_hi)
        return (r_next, accs)

    lax.fori_loop(k_lo, k_hi, p1_chunk, (0, id_accs))

    @pl.when(w < NW - 1)
    def _():
        pltpu.sync_copy(diam_v.at[pl.ds(0, SPW)],
                        diam_hbm.at[pl.ds(seg0, SPW)])

    @pl.when(w == NW - 1)
    def _():
        pltpu.sync_copy(diam_v.at[pl.ds(0, S_LAST)],
                        diam_hbm.at[pl.ds((NW - 1) * SPW, S_LAST)])

    def p2_chunk(k, r_cur):
        base = k * C
        lim = base + C
        pltpu.sync_copy(pos_hbm.at[pl.ds(base, C)], row_v)
        r_hi = find_hi(lim, r_cur)

        def run_body(r, z):
            b0 = _sload(bnd_v, r)
            b1 = _sload(bnd_v, r + 1)
            ls = _sload(sid_v, r) - seg0
            svec = ones / (diam_v[pl.ds(ls, L)] + jnp.float32(0.01))
            scale = jnp.broadcast_to(svec[0], (L,))
            means = [mean_v[pl.ds(ls * D + j * L, L)] for j in range(NJ)]
            rs = jnp.maximum(b0 - base, 0)
            re = jnp.minimum(b1 - base, C)

            @functools.partial(plsc.parallel_loop, rs, re, unroll=4)
            def rownorm(i):
                for j in range(NJ):
                    sl = pl.ds(j * L, L)
                    row_v[i, sl] = (row_v[i, sl] - means[j]) * scale

            return z

        lax.fori_loop(r_cur, r_hi, run_body, 0)

        full = jnp.logical_and(r0 <= base, lim <= r1)

        @pl.when(full)
        def _():
            pltpu.sync_copy(row_v, out_hbm.at[pl.ds(base, C)])

        @pl.when(jnp.logical_not(full))
        def _():
            def wr(i, zz):
                pltpu.sync_copy(row_v.at[i], out_hbm.at[base + i])
                return zz

            lax.fori_loop(jnp.maximum(r0 - base, 0),
                          jnp.minimum(r1 - base, C), wr, 0)

        return jnp.where(_sload(bnd_v, r_hi) > lim, r_hi - 1, r_hi)

    lax.fori_loop(k_lo, k_hi, p2_chunk, 0)


def kernel(pos, idx):
    seg_edges = jnp.minimum(
        jnp.arange(NW + 1, dtype=jnp.int32) * SPW, S).astype(jnp.int32)
    bounds = jnp.searchsorted(idx, seg_edges, side="left").astype(jnp.int32)
    bounds = jnp.concatenate([bounds, jnp.zeros((15,), jnp.int32)])
    pos_out, diam = _seg_unit_norm(pos, idx, bounds)
    return (pos_out, diam)


# speedup vs baseline: 11.8765x; 1.0080x over previous
import functools

import jax
import jax.numpy as jnp
from jax import lax
from jax.experimental import pallas as pl
from jax.experimental.pallas import tpu as pltpu
from jax.experimental.pallas import tpu_sc as plsc

N = 320000
D = 128
S = 10000

L = 16
NJ = D // L
NW = 32
SPW = 320
S_LAST = S - (NW - 1) * SPW
C = 256
CI = 2000
NRUN = SPW + 2 * L

_mesh = plsc.VectorSubcoreMesh(core_axis_name="c", subcore_axis_name="s")


@functools.partial(
    pl.kernel,
    out_type=(
        jax.ShapeDtypeStruct((N, D), jnp.float32),
        jax.ShapeDtypeStruct((S,), jnp.float32),
    ),
    mesh=_mesh,
    compiler_params=pltpu.CompilerParams(needs_layout_passes=False),
    scratch_types=[
        pltpu.VMEM((C, D), jnp.float32),
        pltpu.VMEM((C, D), jnp.float32),
        pltpu.VMEM((CI + L,), jnp.int32),
        pltpu.VMEM((48,), jnp.int32),
        pltpu.VMEM((SPW * D,), jnp.float32),
        pltpu.VMEM((SPW + L,), jnp.float32),
        pltpu.VMEM((NRUN,), jnp.int32),
        pltpu.VMEM((NRUN,), jnp.int32),
    ],
)
def _seg_unit_norm(pos_hbm, idx_hbm, bounds_hbm, out_hbm, diam_hbm,
                   row_v, out_v, ibuf, bounds_v, mean_v, diam_v, bnd_v, sid_v):
    w = lax.axis_index("s") * 2 + lax.axis_index("c")

    def _sload(ref, i):
        return ref[pl.ds(i, L)][0]

    pltpu.sync_copy(bounds_hbm, bounds_v)
    r0 = _sload(bounds_v, w)
    r1 = _sload(bounds_v, w + 1)
    seg0 = w * SPW

    inf = jnp.float32(jnp.inf)
    ones = jnp.ones((L,), jnp.float32)
    lane_iota = lax.iota(jnp.int32, L)
    lane0 = lane_iota == 0
    id_min = jnp.full((L,), inf, jnp.float32)
    id_max = jnp.full((L,), -inf, jnp.float32)
    id_sum = jnp.zeros((L,), jnp.float32)
    id_accs = ((id_min,) * NJ, (id_max,) * NJ, (id_sum,) * NJ)

    def init_diam(t, z):
        diam_v[pl.ds(t * L, L)] = id_max
        return z

    lax.fori_loop(0, (SPW + L) // L, init_diam, 0)

    def p0_chunk(k, wpos):
        base = k * CI
        pltpu.sync_copy(idx_hbm.at[pl.ds(base, CI)], ibuf.at[pl.ds(L, CI)])

        @pl.when(k > 0)
        def _():
            pltpu.sync_copy(idx_hbm.at[pl.ds(base - L, L)],
                            ibuf.at[pl.ds(0, L)])

        @pl.when(k == 0)
        def _():
            ibuf[pl.ds(0, L)] = jnp.full((L,), -1, jnp.int32)

        def group(g, wp):
            off = L + g * L
            v = ibuf[pl.ds(off, L)]
            p = ibuf[pl.ds(off - 1, L)]
            rowv = jnp.broadcast_to(base + g * L, (L,)) + lane_iota
            m = (v != p) & (rowv >= r0) & (rowv < r1)
            plsc.store_compressed(bnd_v.at[pl.ds(wp, L)], rowv, mask=m)
            plsc.store_compressed(sid_v.at[pl.ds(wp, L)], v, mask=m)
            return wp + plsc.all_reduce_population_count(m)[0]

        return lax.fori_loop(0, CI // L, group, wpos)

    nruns = lax.fori_loop(r0 // CI, (r1 + CI - 1) // CI, p0_chunk, 0)
    bnd_v[pl.ds(nruns, L)] = jnp.broadcast_to(r1, (L,))

    def find_hi(lim, lo0):
        def bs(t, lohi):
            lo, hi = lohi
            mid = (lo + hi) // 2
            c = _sload(bnd_v, mid) < lim
            return (jnp.where(c, mid + 1, lo), jnp.where(c, hi, mid))

        lo, _ = lax.fori_loop(0, 9, bs, (lo0, nruns))
        return jnp.minimum(lo, nruns)

    k_lo = r0 // C
    k_hi = (r1 + C - 1) // C

    def flush(s, accs, cnt):
        mins, maxs, sums = accs
        ls = s - seg0
        cntv = jnp.broadcast_to(cnt, (L,)).astype(jnp.float32)
        rcp = ones / jnp.maximum(cntv, ones)
        for j in range(NJ):
            mean_v[pl.ds(ls * D + j * L, L)] = sums[j] * rcp
        dv = maxs[0] - mins[0]
        for j in range(1, NJ):
            dv = jnp.maximum(dv, maxs[j] - mins[j])
        dred = jnp.broadcast_to(jnp.max(dv), (L,))
        plsc.store_scatter(diam_v, [jnp.broadcast_to(ls, (L,))], dred,
                           mask=lane0)

    def p1_chunk(k, carry):
        r_cur, accs = carry
        base = k * C
        lim = base + C
        pltpu.sync_copy(pos_hbm.at[pl.ds(base, C)], row_v)
        r_hi = find_hi(lim, r_cur)

        def run_body(r, accs_):
            b0 = _sload(bnd_v, r)
            b1 = _sload(bnd_v, r + 1)
            rs = jnp.maximum(b0 - base, 0)
            re = jnp.minimum(b1 - base, C)

            def rowacc(i, a):
                mins, maxs, sums = a
                rows = [row_v[i, pl.ds(j * L, L)] for j in range(NJ)]
                mins = tuple(jnp.minimum(m, x) for m, x in zip(mins, rows))
                maxs = tuple(jnp.maximum(m, x) for m, x in zip(maxs, rows))
                sums = tuple(s + x for s, x in zip(sums, rows))
                return (mins, maxs, sums)

            accs_ = plsc.parallel_loop(rs, re, unroll=1,
                                       carry=accs_)(rowacc)

            def fin(_):
                flush(_sload(sid_v, r), accs_, b1 - b0)
                return id_accs

            def keep(_):
                return accs_

            return lax.cond(b1 <= lim, fin, keep, 0)

        accs = lax.fori_loop(r_cur, r_hi, run_body, accs)
        r_next = jnp.where(_sload(bnd_v, r_hi) > lim, r_hi - 1, r_hi)
        return (r_next, accs)

    lax.fori_loop(k_lo, k_hi, p1_chunk, (0, id_accs))

    @pl.when(w < NW - 1)
    def _():
        pltpu.sync_copy(diam_v.at[pl.ds(0, SPW)],
                        diam_hbm.at[pl.ds(seg0, SPW)])

    @pl.when(w == NW - 1)
    def _():
        pltpu.sync_copy(diam_v.at[pl.ds(0, S_LAST)],
                        diam_hbm.at[pl.ds((NW - 1) * SPW, S_LAST)])

    def p2_chunk(k, r_cur):
        base = k * C
        lim = base + C
        pltpu.sync_copy(pos_hbm.at[pl.ds(base, C)], row_v)
        r_hi = find_hi(lim, r_cur)

        def run_body(r, z):
            b0 = _sload(bnd_v, r)
            b1 = _sload(bnd_v, r + 1)
            ls = _sload(sid_v, r) - seg0
            svec = ones / (diam_v[pl.ds(ls, L)] + jnp.float32(0.01))
            scale = jnp.broadcast_to(svec[0], (L,))
            means = [mean_v[pl.ds(ls * D + j * L, L)] for j in range(NJ)]
            rs = jnp.maximum(b0 - base, 0)
            re = jnp.minimum(b1 - base, C)

            @functools.partial(plsc.parallel_loop, rs, re, unroll=1)
            def rownorm(i):
                for j in range(NJ):
                    sl = pl.ds(j * L, L)
                    out_v[i, sl] = (row_v[i, sl] - means[j]) * scale

            return z

        lax.fori_loop(r_cur, r_hi, run_body, 0)

        full = jnp.logical_and(r0 <= base, lim <= r1)

        @pl.when(full)
        def _():
            pltpu.sync_copy(out_v, out_hbm.at[pl.ds(base, C)])

        @pl.when(jnp.logical_not(full))
        def _():
            def wr(i, zz):
                pltpu.sync_copy(out_v.at[i], out_hbm.at[base + i])
                return zz

            lax.fori_loop(jnp.maximum(r0 - base, 0),
                          jnp.minimum(r1 - base, C), wr, 0)

        return jnp.where(_sload(bnd_v, r_hi) > lim, r_hi - 1, r_hi)

    lax.fori_loop(k_lo, k_hi, p2_chunk, 0)


def kernel(pos, idx):
    seg_edges = jnp.minimum(
        jnp.arange(NW + 1, dtype=jnp.int32) * SPW, S).astype(jnp.int32)
    bounds = jnp.searchsorted(idx, seg_edges, side="left").astype(jnp.int32)
    bounds = jnp.concatenate([bounds, jnp.zeros((15,), jnp.int32)])
    pos_out, diam = _seg_unit_norm(pos, idx, bounds)
    return (pos_out, diam)


# speedup vs baseline: 13.7354x; 1.1565x over previous
import functools

import jax
import jax.numpy as jnp
from jax import lax
from jax.experimental import pallas as pl
from jax.experimental.pallas import tpu as pltpu
from jax.experimental.pallas import tpu_sc as plsc

N = 320000
D = 128
S = 10000

L = 16
NJ = D // L
NW = 32
SPW = 320
S_LAST = S - (NW - 1) * SPW
C = 128
CI = 2000
NRUN = SPW + 2 * L

_mesh = plsc.VectorSubcoreMesh(core_axis_name="c", subcore_axis_name="s")


@functools.partial(
    pl.kernel,
    out_type=(
        jax.ShapeDtypeStruct((N, D), jnp.float32),
        jax.ShapeDtypeStruct((S,), jnp.float32),
    ),
    mesh=_mesh,
    compiler_params=pltpu.CompilerParams(needs_layout_passes=False),
    scratch_types=[
        pltpu.VMEM((2, C, D), jnp.float32),
        pltpu.VMEM((2, C, D), jnp.float32),
        pltpu.VMEM((CI + L,), jnp.int32),
        pltpu.VMEM((48,), jnp.int32),
        pltpu.VMEM((SPW * D,), jnp.float32),
        pltpu.VMEM((SPW + L,), jnp.float32),
        pltpu.VMEM((NRUN,), jnp.int32),
        pltpu.VMEM((NRUN,), jnp.int32),
        pltpu.SemaphoreType.DMA((2,)),
        pltpu.SemaphoreType.DMA((2,)),
    ],
)
def _seg_unit_norm(pos_hbm, idx_hbm, bounds_hbm, out_hbm, diam_hbm,
                   row_v, out_v, ibuf, bounds_v, mean_v, diam_v, bnd_v,
                   sid_v, ldsem, stsem):
    w = lax.axis_index("s") * 2 + lax.axis_index("c")

    def _sload(ref, i):
        return ref[pl.ds(i, L)][0]

    pltpu.sync_copy(bounds_hbm, bounds_v)
    r0 = _sload(bounds_v, w)
    r1 = _sload(bounds_v, w + 1)
    seg0 = w * SPW

    inf = jnp.float32(jnp.inf)
    ones = jnp.ones((L,), jnp.float32)
    lane_iota = lax.iota(jnp.int32, L)
    lane0 = lane_iota == 0
    id_min = jnp.full((L,), inf, jnp.float32)
    id_max = jnp.full((L,), -inf, jnp.float32)
    id_sum = jnp.zeros((L,), jnp.float32)
    id_accs = ((id_min,) * NJ, (id_max,) * NJ, (id_sum,) * NJ)

    def init_diam(t, z):
        diam_v[pl.ds(t * L, L)] = id_max
        return z

    lax.fori_loop(0, (SPW + L) // L, init_diam, 0)

    def p0_chunk(k, wpos):
        base = k * CI
        pltpu.sync_copy(idx_hbm.at[pl.ds(base, CI)], ibuf.at[pl.ds(L, CI)])

        @pl.when(k > 0)
        def _():
            pltpu.sync_copy(idx_hbm.at[pl.ds(base - L, L)],
                            ibuf.at[pl.ds(0, L)])

        @pl.when(k == 0)
        def _():
            ibuf[pl.ds(0, L)] = jnp.full((L,), -1, jnp.int32)

        def group(g, wp):
            off = L + g * L
            v = ibuf[pl.ds(off, L)]
            p = ibuf[pl.ds(off - 1, L)]
            rowv = jnp.broadcast_to(base + g * L, (L,)) + lane_iota
            m = (v != p) & (rowv >= r0) & (rowv < r1)
            plsc.store_compressed(bnd_v.at[pl.ds(wp, L)], rowv, mask=m)
            plsc.store_compressed(sid_v.at[pl.ds(wp, L)], v, mask=m)
            return wp + plsc.all_reduce_population_count(m)[0]

        return lax.fori_loop(0, CI // L, group, wpos)

    nruns = lax.fori_loop(r0 // CI, (r1 + CI - 1) // CI, p0_chunk, 0)
    bnd_v[pl.ds(nruns, L)] = jnp.broadcast_to(r1, (L,))

    def find_hi(lim, lo0):
        def bs(t, lohi):
            lo, hi = lohi
            mid = (lo + hi) // 2
            c = _sload(bnd_v, mid) < lim
            return (jnp.where(c, mid + 1, lo), jnp.where(c, hi, mid))

        lo, _ = lax.fori_loop(0, 9, bs, (lo0, nruns))
        return jnp.minimum(lo, nruns)

    k_lo = r0 // C
    k_hi = (r1 + C - 1) // C

    def ld_start(k, p):
        pltpu.async_copy(pos_hbm.at[pl.ds(k * C, C)], row_v.at[p],
                         ldsem.at[p])

    def ld_wait(k, p):
        pltpu.make_async_copy(pos_hbm.at[pl.ds(k * C, C)], row_v.at[p],
                              ldsem.at[p]).wait()

    def st_start(k, p):
        pltpu.async_copy(out_v.at[p], out_hbm.at[pl.ds(k * C, C)],
                         stsem.at[p])

    def st_wait(k, p):
        pltpu.make_async_copy(out_v.at[p], out_hbm.at[pl.ds(k * C, C)],
                              stsem.at[p]).wait()

    def full_chunk(c):
        return jnp.logical_and(r0 <= c * C, (c + 1) * C <= r1)

    def flush(s, accs, cnt):
        mins, maxs, sums = accs
        ls = s - seg0
        cntv = jnp.broadcast_to(cnt, (L,)).astype(jnp.float32)
        rcp = ones / jnp.maximum(cntv, ones)
        for j in range(NJ):
            mean_v[pl.ds(ls * D + j * L, L)] = sums[j] * rcp
        dv = maxs[0] - mins[0]
        for j in range(1, NJ):
            dv = jnp.maximum(dv, maxs[j] - mins[j])
        dred = jnp.broadcast_to(jnp.max(dv), (L,))
        plsc.store_scatter(diam_v, [jnp.broadcast_to(ls, (L,))], dred,
                           mask=lane0)

    @pl.when(k_lo < k_hi)
    def _():
        ld_start(k_lo, 0)

    def p1_chunk(k, carry):
        r_cur, accs = carry
        p = (k - k_lo) % 2
        base = k * C
        lim = base + C
        ld_wait(k, p)

        @pl.when(k + 1 < k_hi)
        def _():
            ld_start(k + 1, 1 - p)

        r_hi = find_hi(lim, r_cur)

        def run_body(r, accs_):
            b0 = _sload(bnd_v, r)
            b1 = _sload(bnd_v, r + 1)
            rs = jnp.maximum(b0 - base, 0)
            re = jnp.minimum(b1 - base, C)

            def rowacc(i, a):
                mins, maxs, sums = a
                rows = [row_v[p, i, pl.ds(j * L, L)] for j in range(NJ)]
                mins = tuple(jnp.minimum(m, x) for m, x in zip(mins, rows))
                maxs = tuple(jnp.maximum(m, x) for m, x in zip(maxs, rows))
                sums = tuple(s + x for s, x in zip(sums, rows))
                return (mins, maxs, sums)

            accs_ = plsc.parallel_loop(rs, re, unroll=1,
                                       carry=accs_)(rowacc)

            def fin(_):
                flush(_sload(sid_v, r), accs_, b1 - b0)
                return id_accs

            def keep(_):
                return accs_

            return lax.cond(b1 <= lim, fin, keep, 0)

        accs = lax.fori_loop(r_cur, r_hi, run_body, accs)
        r_next = jnp.where(_sload(bnd_v, r_hi) > lim, r_hi - 1, r_hi)
        return (r_next, accs)

    lax.fori_loop(k_lo, k_hi, p1_chunk, (0, id_accs))

    @pl.when(k_lo < k_hi)
    def _():
        ld_start(k_lo, 0)

    @pl.when(w < NW - 1)
    def _():
        pltpu.sync_copy(diam_v.at[pl.ds(0, SPW)],
                        diam_hbm.at[pl.ds(seg0, SPW)])

    @pl.when(w == NW - 1)
    def _():
        pltpu.sync_copy(diam_v.at[pl.ds(0, S_LAST)],
                        diam_hbm.at[pl.ds((NW - 1) * SPW, S_LAST)])

    def p2_chunk(k, r_cur):
        p = (k - k_lo) % 2
        base = k * C
        lim = base + C
        ld_wait(k, p)

        @pl.when(k + 1 < k_hi)
        def _():
            ld_start(k + 1, 1 - p)

        @pl.when(jnp.logical_and(k - 2 >= k_lo, full_chunk(k - 2)))
        def _():
            st_wait(k - 2, p)

        r_hi = find_hi(lim, r_cur)

        def run_body(r, z):
            b0 = _sload(bnd_v, r)
            b1 = _sload(bnd_v, r + 1)
            ls = _sload(sid_v, r) - seg0
            svec = ones / (diam_v[pl.ds(ls, L)] + jnp.float32(0.01))
            scale = jnp.broadcast_to(svec[0], (L,))
            means = [mean_v[pl.ds(ls * D + j * L, L)] for j in range(NJ)]
            rs = jnp.maximum(b0 - base, 0)
            re = jnp.minimum(b1 - base, C)

            @functools.partial(plsc.parallel_loop, rs, re, unroll=1)
            def rownorm(i):
                for j in range(NJ):
                    sl = pl.ds(j * L, L)
                    out_v[p, i, sl] = (row_v[p, i, sl] - means[j]) * scale

            return z

        lax.fori_loop(r_cur, r_hi, run_body, 0)

        @pl.when(full_chunk(k))
        def _():
            st_start(k, p)

        @pl.when(jnp.logical_not(full_chunk(k)))
        def _():
            def wr(i, zz):
                pltpu.sync_copy(out_v.at[p, i], out_hbm.at[base + i])
                return zz

            lax.fori_loop(jnp.maximum(r0 - base, 0),
                          jnp.minimum(r1 - base, C), wr, 0)

        return jnp.where(_sload(bnd_v, r_hi) > lim, r_hi - 1, r_hi)

    lax.fori_loop(k_lo, k_hi, p2_chunk, 0)

    def drain(kd):
        @pl.when(jnp.logical_and(kd >= k_lo, full_chunk(kd)))
        def _():
            st_wait(kd, (kd - k_lo) % 2)

    drain(k_hi - 1)
    drain(k_hi - 2)


def kernel(pos, idx):
    seg_edges = jnp.minimum(
        jnp.arange(NW + 1, dtype=jnp.int32) * SPW, S).astype(jnp.int32)
    bounds = jnp.searchsorted(idx, seg_edges, side="left").astype(jnp.int32)
    bounds = jnp.concatenate([bounds, jnp.zeros((15,), jnp.int32)])
    pos_out, diam = _seg_unit_norm(pos, idx, bounds)
    return (pos_out, diam)
